# baseline (speedup 1.0000x reference)
import functools
import jax
import jax.numpy as jnp
from jax import lax
from jax.experimental import pallas as pl
from jax.experimental.pallas import tpu as pltpu
from jax.experimental.pallas import tpu_sc as plsc

MAX_OUT = 300
IOU_T = 0.7
SCORE_T = 0.0
PROP_T = 0.5
LANES = 128
L = 16
KPAD = 304
UNROLL = 16


def _decode_body(delta_ref, anch_ref, out_ref):
    B = delta_ref.shape[0]
    for i in range(B):
        a0 = anch_ref[0]
        a1 = anch_ref[1]
        a2 = anch_ref[2]
        a3 = anch_ref[3]
        xa = (a0 + a1) * 0.5
        ya = (a2 + a3) * 0.5
        wa = a1 - a0
        ha = a3 - a2
        tx = delta_ref[i, 0]
        ty = delta_ref[i, 1]
        tw = delta_ref[i, 2]
        th = delta_ref[i, 3]
        x = tx * wa + xa
        y = ty * ha + ya
        w = jnp.exp(tw) * wa
        h = jnp.exp(th) * ha
        out_ref[i, 0] = jnp.clip(x - w * 0.5, 0.0, 1.0)
        out_ref[i, 1] = jnp.clip(x + w * 0.5, 0.0, 1.0)
        out_ref[i, 2] = jnp.clip(y - h * 0.5, 0.0, 1.0)
        out_ref[i, 3] = jnp.clip(y + h * 0.5, 0.0, 1.0)


def _make_sc_nms(N):
    G1 = N // L
    NV1 = G1 // L
    NV2 = (NV1 + L - 1) // L

    mesh = plsc.VectorSubcoreMesh(core_axis_name="c", subcore_axis_name="s")

    @functools.partial(
        pl.kernel, mesh=mesh,
        compiler_params=pltpu.CompilerParams(needs_layout_passes=False),
        out_type=jax.ShapeDtypeStruct((2 * 4 * KPAD,), jnp.float32),
        scratch_types=[
            pltpu.VMEM((N,), jnp.float32),
            pltpu.VMEM((N,), jnp.float32),
            pltpu.VMEM((N,), jnp.float32),
            pltpu.VMEM((N,), jnp.float32),
            pltpu.VMEM((N,), jnp.float32),
            pltpu.VMEM((G1,), jnp.float32),
            pltpu.VMEM((LANES,), jnp.float32),
            pltpu.VMEM((KPAD,), jnp.float32),
            pltpu.VMEM((KPAD,), jnp.float32),
            pltpu.VMEM((KPAD,), jnp.float32),
            pltpu.VMEM((KPAD,), jnp.float32),
        ],
    )
    def sc_nms(scores_hbm, boxes_hbm, out_hbm,
               sc_v, bx0_v, bx1_v, by0_v, by1_v, l1_v, l2_v,
               k0_v, k1_v, k2_v, k3_v):
        cid = lax.axis_index("c")
        sid = lax.axis_index("s")

        lanes = jnp.arange(L, dtype=jnp.int32)
        m0 = lanes == 0
        zf = jnp.zeros((L,), jnp.float32)
        zi = jnp.zeros((L,), jnp.int32)
        BIG = jnp.int32(1 << 24)

        def rot(x, sh):
            return x.at[(lanes + sh) % L].get(mode="promise_in_bounds")

        def allmax(x):
            for sh in (1, 2, 4, 8):
                x = jnp.maximum(x, rot(x, sh))
            return x

        def allmin(x):
            for sh in (1, 2, 4, 8):
                x = jnp.minimum(x, rot(x, sh))
            return x

        @pl.when(sid == 0)
        def _():
            i = cid
            pltpu.sync_copy(scores_hbm.at[pl.ds(i * N, N)], sc_v)
            pltpu.sync_copy(boxes_hbm.at[pl.ds((i * 4 + 0) * N, N)], bx0_v)
            pltpu.sync_copy(boxes_hbm.at[pl.ds((i * 4 + 1) * N, N)], bx1_v)
            pltpu.sync_copy(boxes_hbm.at[pl.ds((i * 4 + 2) * N, N)], by0_v)
            pltpu.sync_copy(boxes_hbm.at[pl.ds((i * 4 + 3) * N, N)], by1_v)

            def l1_body(cc, _):
                base = cc * L
                acc = zf - 1.0
                for r in range(L):
                    off = r * G1 + base
                    v = sc_v[pl.ds(off, L)]
                    v = jnp.where(v > PROP_T, v, -1.0)
                    sc_v[pl.ds(off, L)] = v
                    acc = jnp.maximum(acc, v)
                l1_v[pl.ds(base, L)] = acc
                return 0
            lax.fori_loop(0, NV1, l1_body, 0)

            for vi in range(LANES // L):
                l2_v[pl.ds(vi * L, L)] = zf - 1e9

            def l2_body(h, _):
                msp = allmax(l1_v[pl.ds(h * L, L)])
                plsc.store_scatter(l2_v, [zi + h], msp, mask=m0)
                return 0
            lax.fori_loop(0, NV1, l2_body, 0)

            for t in range(KPAD // L):
                k0_v[pl.ds(t * L, L)] = zf
                k1_v[pl.ds(t * L, L)] = zf
                k2_v[pl.ds(t * L, L)] = zf
                k3_v[pl.ds(t * L, L)] = zf

            def topmax():
                acc = l2_v[pl.ds(0, L)]
                for vi in range(1, NV2):
                    acc = jnp.maximum(acc, l2_v[pl.ds(vi * L, L)])
                return allmax(acc)

            def step(k_sp, nk):
                M = topmax()
                alive = jnp.logical_and(k_sp < MAX_OUT, M > SCORE_T)

                hc = zi + BIG
                for vi in range(NV2):
                    mvi = l2_v[pl.ds(vi * L, L)] == M
                    hc = jnp.minimum(hc, jnp.where(mvi, vi * L + lanes, BIG))
                h = allmin(hc)
                g1 = plsc.load_gather(l1_v, [h * L + lanes])
                c = allmin(jnp.where(g1 == M, h * L + lanes, BIG))
                col = plsc.load_gather(sc_v, [lanes * G1 + c])
                r = allmin(jnp.where(col == M, lanes, BIG))
                j = c * L + r

                wmask = jnp.logical_and(m0, alive)
                plsc.store_scatter(sc_v, [r * G1 + c], zf - 1.0, mask=wmask)
                newc = allmax(plsc.load_gather(sc_v, [lanes * G1 + c]))
                plsc.store_scatter(l1_v, [c], newc, mask=wmask)
                newh = allmax(plsc.load_gather(l1_v, [h * L + lanes]))
                plsc.store_scatter(l2_v, [h], newh, mask=wmask)

                x0 = plsc.load_gather(bx0_v, [j])
                x1 = plsc.load_gather(bx1_v, [j])
                y0 = plsc.load_gather(by0_v, [j])
                y1 = plsc.load_gather(by1_v, [j])

                area = (x1 - x0) * (y1 - y0)

                def iou_body(t, acc):
                    b = t * L
                    kx0 = k0_v[pl.ds(b, L)]
                    kx1 = k1_v[pl.ds(b, L)]
                    ky0 = k2_v[pl.ds(b, L)]
                    ky1 = k3_v[pl.ds(b, L)]
                    iw = jnp.maximum(
                        jnp.minimum(x1, kx1) - jnp.maximum(x0, kx0), 0.0)
                    ih = jnp.maximum(
                        jnp.minimum(y1, ky1) - jnp.maximum(y0, ky0), 0.0)
                    inter = iw * ih
                    areas = (kx1 - kx0) * (ky1 - ky0)
                    iou = inter / (area + areas - inter + 1e-9)
                    return jnp.logical_or(acc, iou > IOU_T)

                acc = lax.fori_loop(0, nk, iou_body, lanes < 0)
                ov = allmax(jnp.where(acc, 1, 0))
                keep = jnp.logical_and(alive, ov == 0)

                kmask = jnp.logical_and(m0, keep)
                plsc.store_scatter(k0_v, [k_sp], x0, mask=kmask)
                plsc.store_scatter(k1_v, [k_sp], x1, mask=kmask)
                plsc.store_scatter(k2_v, [k_sp], y0, mask=kmask)
                plsc.store_scatter(k3_v, [k_sp], y1, mask=kmask)

                k_sp = k_sp + jnp.where(keep, 1, 0)
                return k_sp, jnp.logical_and(alive, k_sp < MAX_OUT)

            def cond(carry):
                _, flag = carry
                return flag > 0

            def body(carry):
                k_sp, _ = carry
                alive = m0
                nk = k_sp[0] // L + 2
                for _ in range(UNROLL):
                    k_sp, alive = step(k_sp, nk)
                return k_sp, jnp.where(alive, 1, 0)[0]

            lax.while_loop(cond, body, (zi, jnp.int32(1)))

            pltpu.sync_copy(k0_v, out_hbm.at[pl.ds((i * 4 + 0) * KPAD, KPAD)])
            pltpu.sync_copy(k1_v, out_hbm.at[pl.ds((i * 4 + 1) * KPAD, KPAD)])
            pltpu.sync_copy(k2_v, out_hbm.at[pl.ds((i * 4 + 2) * KPAD, KPAD)])
            pltpu.sync_copy(k3_v, out_hbm.at[pl.ds((i * 4 + 3) * KPAD, KPAD)])

    return sc_nms


@jax.jit
def kernel(score_map, delta_map, anchors):
    B, H, W, A = score_map.shape
    N = H * W * A
    R = N // LANES
    assert N % LANES == 0 and B == 2

    deltas = delta_map.reshape(B, N, 4).transpose(0, 2, 1).reshape(B, 4, R, LANES)
    anch = anchors.reshape(N, 4).T.reshape(4, R, LANES)

    boxes = pl.pallas_call(
        _decode_body,
        out_shape=jax.ShapeDtypeStruct((B, 4, R, LANES), jnp.float32),
    )(deltas, anch)

    scores_t = (score_map.reshape(B, N // L, L)
                .transpose(0, 2, 1).reshape(B * N))
    boxes_flat = boxes.reshape(B * 4 * N)

    out = _make_sc_nms(N)(scores_t, boxes_flat)
    kept = out.reshape(B, 4, KPAD)[:, :, :MAX_OUT]
    return kept.transpose(0, 2, 1)


# speedup vs baseline: 1.0108x; 1.0108x over previous
import functools
import jax
import jax.numpy as jnp
from jax import lax
from jax.experimental import pallas as pl
from jax.experimental.pallas import tpu as pltpu
from jax.experimental.pallas import tpu_sc as plsc

MAX_OUT = 300
IOU_T = 0.7
SCORE_T = 0.0
PROP_T = 0.5
LANES = 128
L = 16
KPAD = 304
UNROLL = 8


def _decode_body(delta_ref, anch_ref, out_ref):
    B = delta_ref.shape[0]
    for i in range(B):
        a0 = anch_ref[0]
        a1 = anch_ref[1]
        a2 = anch_ref[2]
        a3 = anch_ref[3]
        xa = (a0 + a1) * 0.5
        ya = (a2 + a3) * 0.5
        wa = a1 - a0
        ha = a3 - a2
        tx = delta_ref[i, 0]
        ty = delta_ref[i, 1]
        tw = delta_ref[i, 2]
        th = delta_ref[i, 3]
        x = tx * wa + xa
        y = ty * ha + ya
        w = jnp.exp(tw) * wa
        h = jnp.exp(th) * ha
        out_ref[i, 0] = jnp.clip(x - w * 0.5, 0.0, 1.0)
        out_ref[i, 1] = jnp.clip(x + w * 0.5, 0.0, 1.0)
        out_ref[i, 2] = jnp.clip(y - h * 0.5, 0.0, 1.0)
        out_ref[i, 3] = jnp.clip(y + h * 0.5, 0.0, 1.0)


def _make_sc_nms(N):
    G1 = N // L
    NV1 = G1 // L
    NV2 = (NV1 + L - 1) // L

    mesh = plsc.VectorSubcoreMesh(core_axis_name="c", subcore_axis_name="s")

    @functools.partial(
        pl.kernel, mesh=mesh,
        compiler_params=pltpu.CompilerParams(needs_layout_passes=False),
        out_type=jax.ShapeDtypeStruct((2 * 4 * KPAD,), jnp.float32),
        scratch_types=[
            pltpu.VMEM((N,), jnp.float32),
            pltpu.VMEM((N,), jnp.float32),
            pltpu.VMEM((N,), jnp.float32),
            pltpu.VMEM((N,), jnp.float32),
            pltpu.VMEM((N,), jnp.float32),
            pltpu.VMEM((G1,), jnp.float32),
            pltpu.VMEM((LANES,), jnp.float32),
            pltpu.VMEM((KPAD,), jnp.float32),
            pltpu.VMEM((KPAD,), jnp.float32),
            pltpu.VMEM((KPAD,), jnp.float32),
            pltpu.VMEM((KPAD,), jnp.float32),
        ],
    )
    def sc_nms(scores_hbm, boxes_hbm, out_hbm,
               sc_v, bx0_v, bx1_v, by0_v, by1_v, l1_v, l2_v,
               k0_v, k1_v, k2_v, k3_v):
        cid = lax.axis_index("c")
        sid = lax.axis_index("s")

        lanes = jnp.arange(L, dtype=jnp.int32)
        m0 = lanes == 0
        zf = jnp.zeros((L,), jnp.float32)
        zi = jnp.zeros((L,), jnp.int32)
        BIG = jnp.int32(1 << 24)

        def rot(x, sh):
            return x.at[(lanes + sh) % L].get(mode="promise_in_bounds")

        def allmax(x):
            for sh in (1, 2, 4, 8):
                x = jnp.maximum(x, rot(x, sh))
            return x

        def allmin(x):
            for sh in (1, 2, 4, 8):
                x = jnp.minimum(x, rot(x, sh))
            return x

        @pl.when(sid == 0)
        def _():
            i = cid
            pltpu.sync_copy(scores_hbm.at[pl.ds(i * N, N)], sc_v)
            pltpu.sync_copy(boxes_hbm.at[pl.ds((i * 4 + 0) * N, N)], bx0_v)
            pltpu.sync_copy(boxes_hbm.at[pl.ds((i * 4 + 1) * N, N)], bx1_v)
            pltpu.sync_copy(boxes_hbm.at[pl.ds((i * 4 + 2) * N, N)], by0_v)
            pltpu.sync_copy(boxes_hbm.at[pl.ds((i * 4 + 3) * N, N)], by1_v)

            def l1_body(cc, _):
                base = cc * L
                acc = zf - 1.0
                for r in range(L):
                    off = r * G1 + base
                    v = sc_v[pl.ds(off, L)]
                    v = jnp.where(v > PROP_T, v, -1.0)
                    sc_v[pl.ds(off, L)] = v
                    acc = jnp.maximum(acc, v)
                l1_v[pl.ds(base, L)] = acc
                return 0
            lax.fori_loop(0, NV1, l1_body, 0)

            for vi in range(LANES // L):
                l2_v[pl.ds(vi * L, L)] = zf - 1e9

            def l2_body(h, _):
                msp = allmax(l1_v[pl.ds(h * L, L)])
                plsc.store_scatter(l2_v, [zi + h], msp, mask=m0)
                return 0
            lax.fori_loop(0, NV1, l2_body, 0)

            for t in range(KPAD // L):
                k0_v[pl.ds(t * L, L)] = zf
                k1_v[pl.ds(t * L, L)] = zf
                k2_v[pl.ds(t * L, L)] = zf
                k3_v[pl.ds(t * L, L)] = zf

            def topmax():
                acc = l2_v[pl.ds(0, L)]
                for vi in range(1, NV2):
                    acc = jnp.maximum(acc, l2_v[pl.ds(vi * L, L)])
                return allmax(acc)

            def step(k_sp, nk):
                M = topmax()
                alive = jnp.logical_and(k_sp < MAX_OUT, M > SCORE_T)

                hc = zi + BIG
                for vi in range(NV2):
                    mvi = l2_v[pl.ds(vi * L, L)] == M
                    hc = jnp.minimum(hc, jnp.where(mvi, vi * L + lanes, BIG))
                h = allmin(hc)
                g1 = plsc.load_gather(l1_v, [h * L + lanes])
                c = allmin(jnp.where(g1 == M, h * L + lanes, BIG))
                col = plsc.load_gather(sc_v, [lanes * G1 + c])
                r = allmin(jnp.where(col == M, lanes, BIG))
                j = c * L + r

                wmask = jnp.logical_and(m0, alive)
                plsc.store_scatter(sc_v, [r * G1 + c], zf - 1.0, mask=wmask)
                newc = allmax(plsc.load_gather(sc_v, [lanes * G1 + c]))
                plsc.store_scatter(l1_v, [c], newc, mask=wmask)
                newh = allmax(plsc.load_gather(l1_v, [h * L + lanes]))
                plsc.store_scatter(l2_v, [h], newh, mask=wmask)

                x0 = plsc.load_gather(bx0_v, [j])
                x1 = plsc.load_gather(bx1_v, [j])
                y0 = plsc.load_gather(by0_v, [j])
                y1 = plsc.load_gather(by1_v, [j])

                area = (x1 - x0) * (y1 - y0)

                def iou_body(t, acc):
                    b = t * L
                    kx0 = k0_v[pl.ds(b, L)]
                    kx1 = k1_v[pl.ds(b, L)]
                    ky0 = k2_v[pl.ds(b, L)]
                    ky1 = k3_v[pl.ds(b, L)]
                    iw = jnp.maximum(
                        jnp.minimum(x1, kx1) - jnp.maximum(x0, kx0), 0.0)
                    ih = jnp.maximum(
                        jnp.minimum(y1, ky1) - jnp.maximum(y0, ky0), 0.0)
                    inter = iw * ih
                    areas = (kx1 - kx0) * (ky1 - ky0)
                    iou = inter / (area + areas - inter + 1e-9)
                    return jnp.logical_or(acc, iou > IOU_T)

                acc = lax.fori_loop(0, nk, iou_body, lanes < 0)
                ov = allmax(jnp.where(acc, 1, 0))
                keep = jnp.logical_and(alive, ov == 0)

                kmask = jnp.logical_and(m0, keep)
                plsc.store_scatter(k0_v, [k_sp], x0, mask=kmask)
                plsc.store_scatter(k1_v, [k_sp], x1, mask=kmask)
                plsc.store_scatter(k2_v, [k_sp], y0, mask=kmask)
                plsc.store_scatter(k3_v, [k_sp], y1, mask=kmask)

                k_sp = k_sp + jnp.where(keep, 1, 0)
                return k_sp, jnp.logical_and(alive, k_sp < MAX_OUT)

            def cond(carry):
                _, flag = carry
                return flag > 0

            def body(carry):
                k_sp, _ = carry
                alive = m0
                nk = k_sp[0] // L + 2
                for _ in range(UNROLL):
                    k_sp, alive = step(k_sp, nk)
                return k_sp, jnp.where(alive, 1, 0)[0]

            lax.while_loop(cond, body, (zi, jnp.int32(1)))

            pltpu.sync_copy(k0_v, out_hbm.at[pl.ds((i * 4 + 0) * KPAD, KPAD)])
            pltpu.sync_copy(k1_v, out_hbm.at[pl.ds((i * 4 + 1) * KPAD, KPAD)])
            pltpu.sync_copy(k2_v, out_hbm.at[pl.ds((i * 4 + 2) * KPAD, KPAD)])
            pltpu.sync_copy(k3_v, out_hbm.at[pl.ds((i * 4 + 3) * KPAD, KPAD)])

    return sc_nms


@jax.jit
def kernel(score_map, delta_map, anchors):
    B, H, W, A = score_map.shape
    N = H * W * A
    R = N // LANES
    assert N % LANES == 0 and B == 2

    deltas = delta_map.reshape(B, N, 4).transpose(0, 2, 1).reshape(B, 4, R, LANES)
    anch = anchors.reshape(N, 4).T.reshape(4, R, LANES)

    boxes = pl.pallas_call(
        _decode_body,
        out_shape=jax.ShapeDtypeStruct((B, 4, R, LANES), jnp.float32),
    )(deltas, anch)

    scores_t = (score_map.reshape(B, N // L, L)
                .transpose(0, 2, 1).reshape(B * N))
    boxes_flat = boxes.reshape(B * 4 * N)

    out = _make_sc_nms(N)(scores_t, boxes_flat)
    kept = out.reshape(B, 4, KPAD)[:, :, :MAX_OUT]
    return kept.transpose(0, 2, 1)
